# Initial kernel scaffold; baseline (speedup 1.0000x reference)
import jax
import jax.numpy as jnp
from jax.experimental import pallas as pl


def kernel(image, feat, W1, b1, Wc, bc, Wb, bb):
    raise NotImplementedError("write your pallas kernel here")


# speedup vs baseline: 44.5935x; 44.5935x over previous
import math

import jax
import jax.numpy as jnp
from jax.experimental import pallas as pl
from jax.experimental.pallas import tpu as pltpu

_SCALES = (128.0, 256.0, 512.0)
_ASPECT_RATIOS = (0.5, 1.0, 2.0)

_HW = 2500
_HWP = 2560
_NA = 9
_NSEL = 10000
_NOUT = 2000


def _generate_anchors(image_shape, feat_shape):
    grid_h, grid_w = feat_shape[-2], feat_shape[-1]
    image_h, image_w = image_shape[-2], image_shape[-1]
    stride_h = image_h // grid_h
    stride_w = image_w // grid_w
    scales = jnp.asarray(_SCALES, dtype=jnp.float32)
    aspect_ratios = jnp.asarray(_ASPECT_RATIOS, dtype=jnp.float32)
    h_ratios = jnp.sqrt(aspect_ratios)
    w_ratios = 1.0 / h_ratios
    ws = (w_ratios[:, None] * scales[None, :]).reshape(-1)
    hs = (h_ratios[:, None] * scales[None, :]).reshape(-1)
    base_anchors = jnp.round(jnp.stack([-ws, -hs, ws, hs], axis=1) / 2.0)
    shift_x = jnp.arange(0, grid_w, dtype=jnp.int32) * stride_w
    shift_y = jnp.arange(0, grid_h, dtype=jnp.int32) * stride_h
    sx, sy = jnp.meshgrid(shift_x, shift_y, indexing='ij')
    sx = sx.reshape(-1)
    sy = sy.reshape(-1)
    shifts = jnp.stack([sx, sy, sx, sy], axis=1).astype(jnp.float32)
    anchors = (shifts[:, None, :] + base_anchors[None, :, :]).reshape(-1, 4)
    return anchors


def _conv3x3_body(a_ref, w_ref, b_ref, o_ref):
    acc = jax.lax.dot_general(
        a_ref[...], w_ref[...], (((1,), (0,)), ((), ())),
        preferred_element_type=jnp.float32)
    o_ref[...] = jnp.maximum(acc + b_ref[...], 0.0)


def _conv3x3_relu(feat, W1, b1):
    x = jnp.moveaxis(feat[0], 0, -1)
    xp = jnp.pad(x, ((1, 1), (1, 1), (0, 0)))
    taps = [xp[ky:ky + 50, kx:kx + 50, :].reshape(_HW, 512)
            for ky in range(3) for kx in range(3)]
    A = jnp.pad(jnp.concatenate(taps, axis=1), ((0, _HWP - _HW), (0, 0)))
    W = jnp.transpose(W1, (2, 3, 1, 0)).reshape(9 * 512, 512)
    mb = 512
    return pl.pallas_call(
        _conv3x3_body, grid=(_HWP // mb,),
        in_specs=[pl.BlockSpec((mb, 9 * 512), lambda i: (i, 0)),
                  pl.BlockSpec((9 * 512, 512), lambda i: (0, 0)),
                  pl.BlockSpec((1, 512), lambda i: (0, 0))],
        out_specs=pl.BlockSpec((mb, 512), lambda i: (i, 0)),
        out_shape=jax.ShapeDtypeStruct((_HWP, 512), jnp.float32),
    )(A, W, b1[None, :])


def _heads_body(a_ref, wc_ref, wb_ref, c_ref, b_ref):
    a = a_ref[...]
    c_ref[...] = jax.lax.dot_general(
        a, wc_ref[...], (((1,), (0,)), ((), ())),
        preferred_element_type=jnp.float32)
    b_ref[...] = jax.lax.dot_general(
        a, wb_ref[...], (((1,), (0,)), ((), ())),
        preferred_element_type=jnp.float32)


def _conv_heads(rpn2d, Wc, Wb):
    Wc_p = jnp.pad(Wc[:, :, 0, 0].T, ((0, 0), (0, 128 - _NA)))
    Wb_p = jnp.pad(Wb[:, :, 0, 0].T, ((0, 0), (0, 128 - 36)))
    return pl.pallas_call(
        _heads_body,
        out_shape=(jax.ShapeDtypeStruct((_HWP, 128), jnp.float32),
                   jax.ShapeDtypeStruct((_HWP, 128), jnp.float32)),
    )(rpn2d, Wc_p, Wb_p)


_BBOX_CLIP = math.log(1000.0 / 16.0)


def _reg_body(pred_ref, anch_ref, cls_ref, prop_ref, score_ref):
    a0 = anch_ref[0:1, :]
    a1 = anch_ref[1:2, :]
    a2 = anch_ref[2:3, :]
    a3 = anch_ref[3:4, :]
    w = a2 - a0
    h = a3 - a1
    cx = a0 + 0.5 * w
    cy = a1 + 0.5 * h
    dx = pred_ref[0:1, :]
    dy = pred_ref[1:2, :]
    dw = jnp.minimum(pred_ref[2:3, :], _BBOX_CLIP)
    dh = jnp.minimum(pred_ref[3:4, :], _BBOX_CLIP)
    pcx = dx * w + cx
    pcy = dy * h + cy
    pw = jnp.exp(dw) * w
    ph = jnp.exp(dh) * h
    prop_ref[0:1, :] = pcx - 0.5 * pw
    prop_ref[1:2, :] = pcy - 0.5 * ph
    prop_ref[2:3, :] = pcx + 0.5 * pw
    prop_ref[3:4, :] = pcy + 0.5 * ph
    score_ref[...] = jax.nn.sigmoid(cls_ref[...])


def _regress_and_score(bbox_t, anchors_t, cls_row):
    np_ = bbox_t.shape[1]
    return pl.pallas_call(
        _reg_body,
        out_shape=(jax.ShapeDtypeStruct((4, np_), jnp.float32),
                   jax.ShapeDtypeStruct((1, np_), jnp.float32)),
    )(bbox_t, anchors_t, cls_row)


_NMS_B = 1024


def _iou_block(rows, tcols):
    x1a = rows[:, 0:1]
    y1a = rows[:, 1:2]
    x2a = rows[:, 2:3]
    y2a = rows[:, 3:4]
    x1b = tcols[0:1, :]
    y1b = tcols[1:2, :]
    x2b = tcols[2:3, :]
    y2b = tcols[3:4, :]
    area_a = (x2a - x1a) * (y2a - y1a)
    area_b = (x2b - x1b) * (y2b - y1b)
    xx1 = jnp.maximum(x1a, x1b)
    yy1 = jnp.maximum(y1a, y1b)
    xx2 = jnp.minimum(x2a, x2b)
    yy2 = jnp.minimum(y2a, y2b)
    inter = jnp.maximum(xx2 - xx1, 0.0) * jnp.maximum(yy2 - yy1, 0.0)
    iou = inter / (area_a + area_b - inter + 1e-9)
    return (iou > 0.7).astype(jnp.float32)


def _nms_blocked_body(rows_ref, tcols_ref, supp_ref, keep_ref, mjj_ref):
    B = _NMS_B
    nb = rows_ref.shape[0] // B

    def block_step(j, _):
        jb = j * B
        rows_j = rows_ref[pl.ds(jb, B), :]
        tcols_j = tcols_ref[:, pl.ds(jb, B)]

        def cross(i, acc):
            ib = i * B
            rows_i = rows_ref[pl.ds(ib, B), :]
            m = _iou_block(rows_i, tcols_j)
            keep_i = keep_ref[pl.ds(i, 1), :]
            return acc + jnp.dot(keep_i, m, preferred_element_type=jnp.float32)

        ext = jax.lax.fori_loop(0, j, cross, jnp.zeros((1, B), jnp.float32))
        supp_j0 = (ext > 0.0).astype(jnp.float32)

        mjj = _iou_block(rows_j, tcols_j)
        ids = jax.lax.broadcasted_iota(jnp.int32, (B, B), 0)
        jds = jax.lax.broadcasted_iota(jnp.int32, (B, B), 1)
        mjj_ref[...] = jnp.where(jds > ids, mjj, 0.0)

        lane = jax.lax.broadcasted_iota(jnp.int32, (1, B), 1)

        def intra(r, s):
            bit = jnp.sum(jnp.where(lane == r, s, 0.0), axis=1, keepdims=True)
            row = mjj_ref[pl.ds(r, 1), :]
            return jnp.maximum(s, row * (1.0 - bit))

        supp_j = jax.lax.fori_loop(0, B, intra, supp_j0)
        keep_ref[pl.ds(j, 1), :] = 1.0 - supp_j
        supp_ref[pl.ds(j, 1), :] = supp_j
        return 0

    jax.lax.fori_loop(0, nb, block_step, 0)


def _nms_suppress_mask(boxes):
    n = boxes.shape[0]
    B = _NMS_B
    n_pad = ((n + B - 1) // B) * B
    nb = n_pad // B
    rows = jnp.zeros((n_pad, 4), jnp.float32).at[:n].set(boxes)
    tcols = rows.T
    supp = pl.pallas_call(
        _nms_blocked_body,
        out_shape=jax.ShapeDtypeStruct((nb, B), jnp.float32),
        scratch_shapes=[
            pltpu.VMEM((nb, B), jnp.float32),
            pltpu.VMEM((B, B), jnp.float32),
        ],
    )(rows, tcols)
    return supp.reshape(-1)[:n] > 0.0


def kernel(image, feat, W1, b1, Wc, bc, Wb, bb):
    image_shape = image.shape
    assert image_shape[-2] == image_shape[-1] == 800
    n_all = _HW * _NA
    n_pad = 22528

    rpn2d = _conv3x3_relu(feat, W1, b1)
    cls2d, bbox2d = _conv_heads(rpn2d, Wc, Wb)

    cls_flat = (cls2d[:_HW, :_NA] + bc[None, :]).reshape(-1)
    bbox_buf = (bbox2d[:_HW, :36] + bb[None, :]).reshape(-1)
    bbox_t = jnp.pad(
        bbox_buf.reshape(_NA, 4, _HW).transpose(1, 2, 0).reshape(4, n_all),
        ((0, 0), (0, n_pad - n_all)))
    anchors = _generate_anchors(image_shape, feat.shape)
    anchors_t = jnp.pad(anchors.T, ((0, 0), (0, n_pad - n_all)))
    cls_row = jnp.pad(cls_flat[None, :], ((0, 0), (0, n_pad - n_all)))

    prop_t, score_row = _regress_and_score(bbox_t, anchors_t, cls_row)
    proposals = prop_t[:, :n_all].T
    scores = score_row[0, :n_all]

    top_scores, top_idx = jax.lax.top_k(scores, _NSEL)
    boxes_sorted = jnp.clip(proposals[top_idx], 0.0, float(image_shape[-1]))
    supp = _nms_suppress_mask(boxes_sorted)
    masked = jnp.where(supp, -jnp.inf, top_scores)
    keep_scores, keep_idx = jax.lax.top_k(masked, _NOUT)
    valid = keep_scores > -jnp.inf
    out_boxes = jnp.where(valid[:, None], boxes_sorted[keep_idx], 0.0)
    out_scores = jnp.where(valid, keep_scores, 0.0)
    return out_boxes, out_scores
